# baseline (speedup 1.0000x reference)
import functools

import jax
import jax.numpy as jnp
from jax.experimental import pallas as pl
from jax.experimental.pallas import tpu as pltpu

_N = 2048
_TI = 512
_NSTEPS = _N // _TI
_LOG2E = 1.4426950408889634
_NEG2 = -9e15 * _LOG2E


def _expm1_neg(x):
    t = x * (1.0 + x / 2.0 * (1.0 + x / 3.0 * (1.0 + x / 4.0 *
        (1.0 + x / 5.0 * (1.0 + x / 6.0 * (1.0 + x / 7.0))))))
    return jnp.where(x > -0.25, t, jnp.exp(x) - 1.0)


def _mm_bf16(a, b):
    return jnp.dot(a.astype(jnp.bfloat16), b.astype(jnp.bfloat16),
                   preferred_element_type=jnp.float32)


def _proj_body(xin, W_ref, asrc_ref, adst_ref, hb_scr, fs_scr,
               fdT_scr, H, O):
    h = _mm_bf16(xin, W_ref[...])
    hb_scr[...] = h.astype(jnp.bfloat16)
    fd_cols = []
    for hh in range(H):
        hs = h[:, hh * O:(hh + 1) * O]
        fs_scr[:, hh:hh + 1] = _LOG2E * jnp.sum(
            hs * asrc_ref[hh:hh + 1, :], axis=1, keepdims=True)
        fd_cols.append(jnp.sum(hs * adst_ref[hh:hh + 1, :], axis=1,
                               keepdims=True))
    fdT_scr[...] = _LOG2E * jnp.concatenate(fd_cols, axis=1).T


def _attn_rows(row0, adj_t, hb_scr, fs_scr, fdT_scr, H, O, out_write):
    madj = jnp.where(adj_t > 0.0, 0.0, _NEG2).astype(jnp.float32)
    for hh in range(H):
        fs = fs_scr[pl.ds(row0, _TI), hh:hh + 1]
        fd = fdT_scr[hh:hh + 1, :]
        s0 = fs + fd
        s = jnp.maximum(s0, 0.2 * s0) + madj
        m = jnp.max(s, axis=1, keepdims=True)
        p = jnp.exp2(s - m)
        denom = jnp.sum(p, axis=1, keepdims=True)
        acc = jnp.dot((p / denom).astype(jnp.bfloat16),
                      hb_scr[:, hh * O:(hh + 1) * O],
                      preferred_element_type=jnp.float32)
        out_write(hh, jnp.where(acc > 0.0, acc, _expm1_neg(acc)))


def _mega_kernel(x_ref, adj_ref, W1_ref, a1s_ref, a1d_ref,
                 W2_ref, a2s_ref, a2d_ref, wr_ref, wd_ref, ret_ref,
                 h1pb, f1s, f1dT, h1, h2pb, f2s, f2dT, h2,
                 *, H1, O1, H2, O2):
    i = pl.program_id(0)

    @pl.when(i == 0)
    def _():
        _proj_body(x_ref[...], W1_ref, a1s_ref, a1d_ref, h1pb,
                   f1s, f1dT, H1, O1)

    @pl.when((i >= 1) & (i <= _NSTEPS))
    def _():
        row0 = (i - 1) * _TI

        def write(hh, val):
            h1[pl.ds(row0, _TI), hh * O1:(hh + 1) * O1] = (
                val.astype(jnp.bfloat16))

        _attn_rows(row0, adj_ref[...], h1pb, f1s, f1dT, H1, O1, write)

    @pl.when(i == _NSTEPS + 1)
    def _():
        _proj_body(h1[...], W2_ref, a2s_ref, a2d_ref,
                   h2pb, f2s, f2dT, H2, O2)

    @pl.when(i >= _NSTEPS + 2)
    def _():
        row0 = (i - _NSTEPS - 2) * _TI

        def write(hh, val):
            h2[pl.ds(row0, _TI), hh * O2:(hh + 1) * O2] = (
                val.astype(jnp.bfloat16))

        _attn_rows(row0, adj_ref[...], h2pb, f2s, f2dT, H2, O2, write)

    @pl.when(i == 2 * _NSTEPS + 1)
    def _():
        half = _N // 2
        rna = jnp.dot(h2[0:half, :], wr_ref[...].astype(jnp.bfloat16),
                      preferred_element_type=jnp.float32)
        dis = jnp.dot(h2[half:_N, :], wd_ref[...].astype(jnp.bfloat16),
                      preferred_element_type=jnp.float32)
        ret_ref[...] = jax.lax.dot_general(
            rna.astype(jnp.bfloat16), dis.astype(jnp.bfloat16),
            (((1,), (1,)), ((), ())),
            preferred_element_type=jnp.float32)


def kernel(x, adj, W1, a1_src, a1_dst, W2, a2_src, a2_dst, Wd_rna, Wd_dis):
    n, d_in = x.shape
    H1, O1 = a1_src.shape
    H2, O2 = a2_src.shape
    d1 = H1 * O1
    d2 = H2 * O2
    half = n // 2

    full = lambda shape: pl.BlockSpec(shape, lambda i: (0, 0))

    def adj_map(i):
        j = jnp.where(i <= _NSTEPS, i - 1, i - _NSTEPS - 2)
        return jnp.clip(j, 0, _NSTEPS - 1), 0

    ret = pl.pallas_call(
        functools.partial(_mega_kernel, H1=H1, O1=O1, H2=H2, O2=O2),
        grid=(2 * _NSTEPS + 2,),
        in_specs=[
            full((n, d_in)),
            pl.BlockSpec((_TI, n), adj_map),
            full((d_in, d1)),
            full((H1, O1)),
            full((H1, O1)),
            full((d1, d2)),
            full((H2, O2)),
            full((H2, O2)),
            full((d2, Wd_rna.shape[1])),
            full((d2, Wd_dis.shape[1])),
        ],
        out_specs=full((half, half)),
        out_shape=jax.ShapeDtypeStruct((half, half), jnp.float32),
        scratch_shapes=[
            pltpu.VMEM((n, d1), jnp.bfloat16),
            pltpu.VMEM((n, H1), jnp.float32),
            pltpu.VMEM((H1, n), jnp.float32),
            pltpu.VMEM((n, d1), jnp.bfloat16),
            pltpu.VMEM((n, d2), jnp.bfloat16),
            pltpu.VMEM((n, H2), jnp.float32),
            pltpu.VMEM((H2, n), jnp.float32),
            pltpu.VMEM((n, d2), jnp.bfloat16),
        ],
    )(x, adj, W1, a1_src, a1_dst, W2, a2_src, a2_dst, Wd_rna, Wd_dis)

    return ret.reshape(-1)
